# Initial kernel scaffold; baseline (speedup 1.0000x reference)
import functools

import jax
import jax.numpy as jnp
from jax import lax
from jax.experimental import pallas as pl
from jax.experimental.pallas import tpu as pltpu
from jax.experimental.pallas import tpu_sc as plsc

B, S, F = 1024, 50, 26
NSPARSE = F - 1
EMB = 32
N = B * S
OUT_W = 1 + NSPARSE * EMB

NC, NS = 2, 16
NW = NC * NS
PER_W = N // NW
C = 32
CHUNKS = PER_W // C


def _sc_body(ids_hbm, dense_hbm, table_hbm, out_hbm, idx_v, asm_v, dv_v, sem):
    cid = lax.axis_index("c")
    sid = lax.axis_index("s")
    wid = sid * NC + cid

    def chunk(i, carry):
        base = wid * PER_W + i * C
        pltpu.sync_copy(ids_hbm.at[:, pl.ds(base, C)], idx_v)
        pltpu.sync_copy(dense_hbm.at[pl.ds(base, C)], dv_v)
        for g in range(C // 16):
            rows = lax.iota(jnp.int32, 16) + g * 16
            cols = jnp.zeros((16,), jnp.int32)
            plsc.store_scatter(asm_v, [rows, cols], dv_v[pl.ds(g * 16, 16)])
        for j in range(NSPARSE):
            pltpu.async_copy(
                table_hbm.at[idx_v.at[j]],
                asm_v.at[:, pl.ds(1 + EMB * j, EMB)],
                sem,
            ).wait()
        pltpu.sync_copy(asm_v, out_hbm.at[pl.ds(base, C), :])
        return carry

    lax.fori_loop(0, CHUNKS, chunk, 0)


@jax.jit
def _sc_call(ids_t, dense, table):
    return pl.kernel(
        _sc_body,
        out_type=jax.ShapeDtypeStruct((N, OUT_W), jnp.float32),
        mesh=plsc.VectorSubcoreMesh(
            core_axis_name="c", subcore_axis_name="s",
            num_cores=NC, num_subcores=NS,
        ),
        scratch_types=[
            pltpu.VMEM((NSPARSE, C), jnp.int32),
            pltpu.VMEM((C, OUT_W), jnp.float32),
            pltpu.VMEM((C,), jnp.float32),
            pltpu.SemaphoreType.DMA,
        ],
    )(ids_t, dense, table)


def kernel(x, emb0):
    ids_t = x[:, :, 1:].reshape(N, NSPARSE).astype(jnp.int32).T
    dense = x[:, :, 0].reshape(N)
    out = _sc_call(ids_t, dense, emb0)
    return out.reshape(B, S, OUT_W)


# speedup vs baseline: 2.3119x; 2.3119x over previous
import functools

import jax
import jax.numpy as jnp
from jax import lax
from jax.experimental import pallas as pl
from jax.experimental.pallas import tpu as pltpu
from jax.experimental.pallas import tpu_sc as plsc

B, S, F = 1024, 50, 26
NSPARSE = F - 1
EMB = 32
N = B * S
OUT_W = 1 + NSPARSE * EMB

NC, NS = 2, 16
NW = NC * NS
C = 64
NCHUNK = N // C
TRIPS = NCHUNK // NW
G = C
NG = C * NSPARSE // G
L = 16


def _sc_body(ids_hbm, dense_hbm, table_hbm, out_hbm, idx_v, gbuf_v, dv_v,
             asm_v, sem):
    cid = lax.axis_index("c")
    sid = lax.axis_index("s")
    wid = sid * NC + cid

    def chunk(i, carry):
        k = i * NW + wid
        base = k * C
        pltpu.sync_copy(ids_hbm.at[pl.ds(k * NG, NG), :], idx_v)
        pltpu.sync_copy(dense_hbm.at[pl.ds(base, C)], dv_v)
        copies = [
            pltpu.async_copy(
                table_hbm.at[idx_v.at[g]],
                gbuf_v.at[pl.ds(g * G, G), :],
                sem,
            )
            for g in range(NG)
        ]
        for cp in copies:
            cp.wait()

        def pos(p, c2):
            out_base = p * OUT_W + 1
            for r in range(NSPARSE):
                row = p * NSPARSE + r
                asm_v[pl.ds(out_base + r * EMB, L)] = gbuf_v[row, pl.ds(0, L)]
                asm_v[pl.ds(out_base + r * EMB + L, L)] = (
                    gbuf_v[row, pl.ds(L, L)])
            return c2

        lax.fori_loop(0, C, pos, 0)
        iota = lax.iota(jnp.int32, L)
        for g2 in range(C // L):
            plsc.store_scatter(
                asm_v,
                [(iota + g2 * L) * OUT_W],
                dv_v[pl.ds(g2 * L, L)],
            )
        pltpu.sync_copy(asm_v, out_hbm.at[pl.ds(base * OUT_W, C * OUT_W)])
        return carry

    lax.fori_loop(0, TRIPS, chunk, 0)


@jax.jit
def _sc_call(ids_g, dense, table):
    return pl.kernel(
        _sc_body,
        out_type=jax.ShapeDtypeStruct((N * OUT_W,), jnp.float32),
        mesh=plsc.VectorSubcoreMesh(
            core_axis_name="c", subcore_axis_name="s",
            num_cores=NC, num_subcores=NS,
        ),
        scratch_types=[
            pltpu.VMEM((NG, G), jnp.int32),
            pltpu.VMEM((C * NSPARSE, EMB), jnp.float32),
            pltpu.VMEM((C,), jnp.float32),
            pltpu.VMEM((C * OUT_W,), jnp.float32),
            pltpu.SemaphoreType.DMA,
        ],
        compiler_params=pltpu.CompilerParams(
            use_tc_tiling_on_sc=False, needs_layout_passes=False),
    )(ids_g, dense, table)


def kernel(x, emb0):
    ids_g = x[:, :, 1:].astype(jnp.int32).reshape(N * NSPARSE // G, G)
    dense = x[:, :, 0].reshape(N)
    out = _sc_call(ids_g, dense, emb0)
    return out.reshape(B, S, OUT_W)


# speedup vs baseline: 2.7901x; 1.2069x over previous
import functools

import jax
import jax.numpy as jnp
from jax import lax
from jax.experimental import pallas as pl
from jax.experimental.pallas import tpu as pltpu
from jax.experimental.pallas import tpu_sc as plsc

B, S, F = 1024, 50, 26
NSPARSE = F - 1
EMB = 32
OUT_W = 1 + NSPARSE * EMB

NC, NS = 2, 16
NW = NC * NS
TRIPS = B // NW
L = 16


def _sc_body(ids_hbm, dense_hbm, table_hbm, out_hbm, idx_v, gbuf_v, dv_v,
             asm_v, sem):
    cid = lax.axis_index("c")
    sid = lax.axis_index("s")
    wid = sid * NC + cid

    def chunk(i, carry):
        b = i * NW + wid
        pltpu.sync_copy(ids_hbm.at[pl.ds(b * NSPARSE, NSPARSE), :], idx_v)
        pltpu.sync_copy(dense_hbm.at[b], dv_v)
        copies = [
            pltpu.async_copy(
                table_hbm.at[idx_v.at[g]],
                gbuf_v.at[pl.ds(g * S, S), :],
                sem,
            )
            for g in range(NSPARSE)
        ]
        for cp in copies:
            cp.wait()

        def pos(p, c2):
            for r in range(NSPARSE):
                row = r * S + p
                asm_v[p, pl.ds(1 + r * EMB, L)] = gbuf_v[row, pl.ds(0, L)]
                asm_v[p, pl.ds(1 + r * EMB + L, L)] = gbuf_v[row, pl.ds(L, L)]
            return c2

        lax.fori_loop(0, S, pos, 0)
        iota = lax.iota(jnp.int32, L)
        zeros = jnp.zeros((L,), jnp.int32)
        for off in (0, L, 2 * L, S - L):
            rows = iota + off
            plsc.store_scatter(asm_v, [rows, zeros], dv_v[pl.ds(off, L)])
        pltpu.sync_copy(asm_v, out_hbm.at[b])
        return carry

    lax.fori_loop(0, TRIPS, chunk, 0)


@jax.jit
def _sc_call(ids_g, dense, table):
    return pl.kernel(
        _sc_body,
        out_type=jax.ShapeDtypeStruct((B, S, OUT_W), jnp.float32),
        mesh=plsc.VectorSubcoreMesh(
            core_axis_name="c", subcore_axis_name="s",
            num_cores=NC, num_subcores=NS,
        ),
        scratch_types=[
            pltpu.VMEM((NSPARSE, S), jnp.int32),
            pltpu.VMEM((S * NSPARSE, EMB), jnp.float32),
            pltpu.VMEM((S,), jnp.float32),
            pltpu.VMEM((S, OUT_W), jnp.float32),
            pltpu.SemaphoreType.DMA,
        ],
        compiler_params=pltpu.CompilerParams(
            use_tc_tiling_on_sc=False, needs_layout_passes=False),
    )(ids_g, dense, table)


def kernel(x, emb0):
    ids_g = x[:, :, 1:].astype(jnp.int32).transpose(0, 2, 1).reshape(
        B * NSPARSE, S)
    dense = x[:, :, 0]
    out = _sc_call(ids_g, dense, emb0)
    return out


# speedup vs baseline: 2.7987x; 1.0031x over previous
import functools

import jax
import jax.numpy as jnp
from jax import lax
from jax.experimental import pallas as pl
from jax.experimental.pallas import tpu as pltpu
from jax.experimental.pallas import tpu_sc as plsc

B, S, F = 1024, 50, 26
NSPARSE = F - 1
EMB = 32
OUT_W = 1 + NSPARSE * EMB
H = S // 2

NC, NS = 2, 16
NW = NC * NS
TRIPS = B // NW
L = 16


def _sc_body(ids_hbm, dense_hbm, table_hbm, out_hbm,
             idxP, idxT, g0, g1, a0, a1, dv,
             sg0, sg1, so0, so1):
    cid = lax.axis_index("c")
    sid = lax.axis_index("s")
    wid = sid * NC + cid
    iota = lax.iota(jnp.int32, L)
    zeros = jnp.zeros((L,), jnp.int32)

    def assemble(gb, am, half):
        def pos(p, c2):
            for r in range(NSPARSE):
                row = r * H + p
                am[p, pl.ds(1 + r * EMB, L)] = gb[row, pl.ds(0, L)]
                am[p, pl.ds(1 + r * EMB + L, L)] = gb[row, pl.ds(L, L)]
            return c2

        lax.fori_loop(0, H, pos, 0)
        for off in (0, H - L):
            plsc.store_scatter(am, [iota + off, zeros],
                               dv[pl.ds(half * H + off, L)])

    def chunk(i, carry):
        b = i * NW + wid
        pltpu.sync_copy(ids_hbm.at[pl.ds(b * S, S), :], idxP)
        pltpu.sync_copy(dense_hbm.at[b], dv)
        for j in range(NSPARSE):
            col = jnp.full((L,), j, jnp.int32)
            for half in (0, 1):
                for off in (0, H - L):
                    v = plsc.load_gather(idxP, [iota + half * H + off, col])
                    idxT[2 * j + half, pl.ds(off, L)] = v
        gath0 = [
            pltpu.async_copy(table_hbm.at[idxT.at[2 * j]],
                             g0.at[pl.ds(j * H, H), :], sg0)
            for j in range(NSPARSE)
        ]
        gath1 = [
            pltpu.async_copy(table_hbm.at[idxT.at[2 * j + 1]],
                             g1.at[pl.ds(j * H, H), :], sg1)
            for j in range(NSPARSE)
        ]

        @pl.when(i > 0)
        def _():
            pltpu.make_async_copy(a0, out_hbm.at[b, pl.ds(0, H), :],
                                  so0).wait()

        for cp in gath0:
            cp.wait()
        assemble(g0, a0, 0)
        pltpu.async_copy(a0, out_hbm.at[b, pl.ds(0, H), :], so0)

        @pl.when(i > 0)
        def _():
            pltpu.make_async_copy(a1, out_hbm.at[b, pl.ds(H, H), :],
                                  so1).wait()

        for cp in gath1:
            cp.wait()
        assemble(g1, a1, 1)
        pltpu.async_copy(a1, out_hbm.at[b, pl.ds(H, H), :], so1)
        return carry

    lax.fori_loop(0, TRIPS, chunk, 0)
    pltpu.make_async_copy(a0, out_hbm.at[0, pl.ds(0, H), :], so0).wait()
    pltpu.make_async_copy(a1, out_hbm.at[0, pl.ds(H, H), :], so1).wait()


@jax.jit
def _sc_call(ids_g, dense, table):
    return pl.kernel(
        _sc_body,
        out_type=jax.ShapeDtypeStruct((B, S, OUT_W), jnp.float32),
        mesh=plsc.VectorSubcoreMesh(
            core_axis_name="c", subcore_axis_name="s",
            num_cores=NC, num_subcores=NS,
        ),
        scratch_types=[
            pltpu.VMEM((S, NSPARSE), jnp.int32),
            pltpu.VMEM((2 * NSPARSE, H), jnp.int32),
            pltpu.VMEM((H * NSPARSE, EMB), jnp.float32),
            pltpu.VMEM((H * NSPARSE, EMB), jnp.float32),
            pltpu.VMEM((H, OUT_W), jnp.float32),
            pltpu.VMEM((H, OUT_W), jnp.float32),
            pltpu.VMEM((S,), jnp.float32),
            pltpu.SemaphoreType.DMA,
            pltpu.SemaphoreType.DMA,
            pltpu.SemaphoreType.DMA,
            pltpu.SemaphoreType.DMA,
        ],
        compiler_params=pltpu.CompilerParams(
            use_tc_tiling_on_sc=False, needs_layout_passes=False),
    )(ids_g, dense, table)


def kernel(x, emb0):
    ids_g = x[:, :, 1:].astype(jnp.int32).reshape(B * S, NSPARSE)
    dense = x[:, :, 0]
    return _sc_call(ids_g, dense, emb0)


# speedup vs baseline: 2.8790x; 1.0287x over previous
import functools

import jax
import jax.numpy as jnp
from jax import lax
from jax.experimental import pallas as pl
from jax.experimental.pallas import tpu as pltpu
from jax.experimental.pallas import tpu_sc as plsc

B, S, F = 1024, 50, 26
NSPARSE = F - 1
EMB = 32
OUT_W = 1 + NSPARSE * EMB
H = S // 2
IPB = S * NSPARSE

NC, NS = 2, 16
NW = NC * NS
TRIPS = B // NW
L = 16

IDS_ROWS = B * S * NSPARSE // 128
IDS_WIN = 11
DV_ROWS = B * S // 128
DV_WIN = 2


def _sc_body(ids_hbm, dense_hbm, table_hbm, out_hbm,
             idsW, dvW, idxT, g0, g1, a0, a1,
             sg0, sg1, so0, so1):
    cid = lax.axis_index("c")
    sid = lax.axis_index("s")
    wid = sid * NC + cid
    iota = lax.iota(jnp.int32, L)
    zeros = jnp.zeros((L,), jnp.int32)
    posv = {(half, off): (iota + off + half * H) * NSPARSE
            for half in (0, 1) for off in (0, H - L)}
    dposv = {(half, off): iota + off + half * H
             for half in (0, 1) for off in (0, H - L)}

    def assemble(gb, am, half, od):
        def pos(p, c2):
            for r in range(NSPARSE):
                row = r * H + p
                am[p, pl.ds(1 + r * EMB, L)] = gb[row, pl.ds(0, L)]
                am[p, pl.ds(1 + r * EMB + L, L)] = gb[row, pl.ds(L, L)]
            return c2

        lax.fori_loop(0, H, pos, 0)
        for off in (0, H - L):
            fl = dposv[(half, off)] + od
            vals = plsc.load_gather(dvW, [fl >> 7, fl & 127])
            plsc.store_scatter(am, [iota + off, zeros], vals)

    def chunk(i, carry):
        b = i * NW + wid
        r0 = jnp.minimum((b * IPB) >> 7, IDS_ROWS - IDS_WIN)
        o = b * IPB - (r0 << 7)
        r0d = jnp.minimum((b * S) >> 7, DV_ROWS - DV_WIN)
        od = b * S - (r0d << 7)
        pltpu.sync_copy(ids_hbm.at[pl.ds(r0, IDS_WIN), :], idsW)
        pltpu.sync_copy(dense_hbm.at[pl.ds(r0d, DV_WIN), :], dvW)
        for j in range(NSPARSE):
            for half in (0, 1):
                for off in (0, H - L):
                    fl = posv[(half, off)] + (o + j)
                    v = plsc.load_gather(idsW, [fl >> 7, fl & 127])
                    idxT[2 * j + half, pl.ds(off, L)] = v
        gath0 = [
            pltpu.async_copy(table_hbm.at[idxT.at[2 * j]],
                             g0.at[pl.ds(j * H, H), :], sg0)
            for j in range(NSPARSE)
        ]
        gath1 = [
            pltpu.async_copy(table_hbm.at[idxT.at[2 * j + 1]],
                             g1.at[pl.ds(j * H, H), :], sg1)
            for j in range(NSPARSE)
        ]

        @pl.when(i > 0)
        def _():
            pltpu.make_async_copy(a0, out_hbm.at[b, pl.ds(0, H), :],
                                  so0).wait()

        for cp in gath0:
            cp.wait()
        assemble(g0, a0, 0, od)
        pltpu.async_copy(a0, out_hbm.at[b, pl.ds(0, H), :], so0)

        @pl.when(i > 0)
        def _():
            pltpu.make_async_copy(a1, out_hbm.at[b, pl.ds(H, H), :],
                                  so1).wait()

        for cp in gath1:
            cp.wait()
        assemble(g1, a1, 1, od)
        pltpu.async_copy(a1, out_hbm.at[b, pl.ds(H, H), :], so1)
        return carry

    lax.fori_loop(0, TRIPS, chunk, 0)
    pltpu.make_async_copy(a0, out_hbm.at[0, pl.ds(0, H), :], so0).wait()
    pltpu.make_async_copy(a1, out_hbm.at[0, pl.ds(H, H), :], so1).wait()


@jax.jit
def _sc_call(ids_g, dense, table):
    return pl.kernel(
        _sc_body,
        out_type=jax.ShapeDtypeStruct((B, S, OUT_W), jnp.float32),
        mesh=plsc.VectorSubcoreMesh(
            core_axis_name="c", subcore_axis_name="s",
            num_cores=NC, num_subcores=NS,
        ),
        scratch_types=[
            pltpu.VMEM((IDS_WIN, 128), jnp.int32),
            pltpu.VMEM((DV_WIN, 128), jnp.float32),
            pltpu.VMEM((2 * NSPARSE, H), jnp.int32),
            pltpu.VMEM((H * NSPARSE, EMB), jnp.float32),
            pltpu.VMEM((H * NSPARSE, EMB), jnp.float32),
            pltpu.VMEM((H, OUT_W), jnp.float32),
            pltpu.VMEM((H, OUT_W), jnp.float32),
            pltpu.SemaphoreType.DMA,
            pltpu.SemaphoreType.DMA,
            pltpu.SemaphoreType.DMA,
            pltpu.SemaphoreType.DMA,
        ],
        compiler_params=pltpu.CompilerParams(
            use_tc_tiling_on_sc=False, needs_layout_passes=False),
    )(ids_g, dense, table)


def kernel(x, emb0):
    ids_g = x[:, :, 1:].astype(jnp.int32).reshape(IDS_ROWS, 128)
    dense = x[:, :, 0].reshape(DV_ROWS, 128)
    return _sc_call(ids_g, dense, emb0)


# speedup vs baseline: 3.1109x; 1.0806x over previous
import functools

import jax
import jax.numpy as jnp
from jax import lax
from jax.experimental import pallas as pl
from jax.experimental.pallas import tpu as pltpu
from jax.experimental.pallas import tpu_sc as plsc

B, S, F = 1024, 50, 26
NSPARSE = F - 1
EMB = 32
OUT_W = 1 + NSPARSE * EMB
S_PAD = 56
W_PAD = 896
H0 = 24
H1 = S - H0
R1 = S_PAD - H0
IPB = S * NSPARSE

NC, NS = 2, 16
NW = NC * NS
TRIPS = B // NW
L = 16

IDS_ROWS = B * S * NSPARSE // 128
IDS_WIN = 11
DV_ROWS = B * S // 128
DV_WIN = 2


def _sc_body(ids_hbm, dense_hbm, table_hbm, out_hbm,
             idsW, dvW, idxT, g0, g1, a0, a1,
             sg0, sg1, so0, so1):
    cid = lax.axis_index("c")
    sid = lax.axis_index("s")
    wid = sid * NC + cid
    iota = lax.iota(jnp.int32, L)
    zeros = jnp.zeros((L,), jnp.int32)
    blocks = ((0, 0, H0), (1, H0, H1))
    posv = {(blk, off): (iota + off + base) * NSPARSE
            for blk, base, n in blocks for off in (0, n - L)}

    def assemble(gb, am, base, n, od):
        def pos(p, c2):
            for r in range(NSPARSE):
                row = r * n + p
                am[p, pl.ds(1 + r * EMB, L)] = gb[row, pl.ds(0, L)]
                am[p, pl.ds(1 + r * EMB + L, L)] = gb[row, pl.ds(L, L)]
            return c2

        lax.fori_loop(0, n, pos, 0)
        for off in (0, n - L):
            fl = iota + off + base + od
            vals = plsc.load_gather(dvW, [fl >> 7, fl & 127])
            plsc.store_scatter(am, [iota + off, zeros], vals)

    def chunk(i, carry):
        b = i * NW + wid
        r0 = jnp.minimum((b * IPB) >> 7, IDS_ROWS - IDS_WIN)
        o = b * IPB - (r0 << 7)
        r0d = jnp.minimum((b * S) >> 7, DV_ROWS - DV_WIN)
        od = b * S - (r0d << 7)
        pltpu.sync_copy(ids_hbm.at[pl.ds(r0, IDS_WIN), :], idsW)
        pltpu.sync_copy(dense_hbm.at[pl.ds(r0d, DV_WIN), :], dvW)
        for j in range(NSPARSE):
            for blk, base, n in blocks:
                for off in (0, n - L):
                    fl = posv[(blk, off)] + (o + j)
                    v = plsc.load_gather(idsW, [fl >> 7, fl & 127])
                    idxT[2 * j + blk, pl.ds(off, L)] = v
        gath0 = [
            pltpu.async_copy(table_hbm.at[idxT.at[2 * j, pl.ds(0, H0)]],
                             g0.at[pl.ds(j * H0, H0), :], sg0)
            for j in range(NSPARSE)
        ]
        gath1 = [
            pltpu.async_copy(table_hbm.at[idxT.at[2 * j + 1]],
                             g1.at[pl.ds(j * H1, H1), :], sg1)
            for j in range(NSPARSE)
        ]

        @pl.when(i > 0)
        def _():
            pltpu.make_async_copy(a0, out_hbm.at[b, pl.ds(0, H0), :],
                                  so0).wait()

        for cp in gath0:
            cp.wait()
        assemble(g0, a0, 0, H0, od)
        pltpu.async_copy(a0, out_hbm.at[b, pl.ds(0, H0), :], so0)

        @pl.when(i > 0)
        def _():
            pltpu.make_async_copy(a1, out_hbm.at[b, pl.ds(H0, R1), :],
                                  so1).wait()

        for cp in gath1:
            cp.wait()
        assemble(g1, a1, H0, H1, od)
        pltpu.async_copy(a1, out_hbm.at[b, pl.ds(H0, R1), :], so1)
        return carry

    lax.fori_loop(0, TRIPS, chunk, 0)
    pltpu.make_async_copy(a0, out_hbm.at[0, pl.ds(0, H0), :], so0).wait()
    pltpu.make_async_copy(a1, out_hbm.at[0, pl.ds(H0, R1), :], so1).wait()


@jax.jit
def _sc_call(ids_g, dense, table):
    return pl.kernel(
        _sc_body,
        out_type=jax.ShapeDtypeStruct((B, S_PAD, W_PAD), jnp.float32),
        mesh=plsc.VectorSubcoreMesh(
            core_axis_name="c", subcore_axis_name="s",
            num_cores=NC, num_subcores=NS,
        ),
        scratch_types=[
            pltpu.VMEM((IDS_WIN, 128), jnp.int32),
            pltpu.VMEM((DV_WIN, 128), jnp.float32),
            pltpu.VMEM((2 * NSPARSE, H1), jnp.int32),
            pltpu.VMEM((H0 * NSPARSE, EMB), jnp.float32),
            pltpu.VMEM((H1 * NSPARSE, EMB), jnp.float32),
            pltpu.VMEM((H0, W_PAD), jnp.float32),
            pltpu.VMEM((R1, W_PAD), jnp.float32),
            pltpu.SemaphoreType.DMA,
            pltpu.SemaphoreType.DMA,
            pltpu.SemaphoreType.DMA,
            pltpu.SemaphoreType.DMA,
        ],
        compiler_params=pltpu.CompilerParams(
            use_tc_tiling_on_sc=False, needs_layout_passes=False),
    )(ids_g, dense, table)


def kernel(x, emb0):
    ids_g = x[:, :, 1:].astype(jnp.int32).reshape(IDS_ROWS, 128)
    dense = x[:, :, 0].reshape(DV_ROWS, 128)
    out = _sc_call(ids_g, dense, emb0)
    return out[:, :S, :OUT_W]
